# Initial kernel scaffold; baseline (speedup 1.0000x reference)
import jax
import jax.numpy as jnp
from jax.experimental import pallas as pl


def kernel(x, adj_t, edge_weight, W1, b1, W2, b2):
    raise NotImplementedError("write your pallas kernel here")


# speedup vs baseline: 14.4581x; 14.4581x over previous
import functools

import jax
import jax.numpy as jnp
from jax import lax
from jax.experimental import pallas as pl
from jax.experimental.pallas import tpu as pltpu
from jax.experimental.pallas import tpu_sc as plsc

N = 10000
E = 320000
D1 = 128
D2 = 40
D2P = 48

NC = 2
NS = 16
NW = NC * NS
BLK = 128
NBLK = 80
EP = NW * NBLK * BLK
ACC_ROWS = 10240
STRIPE = ACC_ROWS // NS

_mesh = plsc.VectorSubcoreMesh(core_axis_name="c", subcore_axis_name="s")
_sc_params = pltpu.CompilerParams(use_tc_tiling_on_sc=False)
_f32 = jnp.float32


@functools.partial(
    pl.kernel,
    mesh=_mesh,
    compiler_params=_sc_params,
    out_type=(
        jax.ShapeDtypeStruct((NC, ACC_ROWS, 16), _f32),
        jax.ShapeDtypeStruct((NC, ACC_ROWS, 16), _f32),
    ),
    scratch_types=[
        pltpu.VMEM((NBLK, BLK), jnp.int32),
        pltpu.VMEM((NBLK, BLK), _f32),
        pltpu.VMEM((BLK, 16), _f32),
        pltpu.VMEM((BLK, 16), _f32),
        pltpu.VMEM_SHARED((ACC_ROWS, 16), _f32),
        pltpu.VMEM_SHARED((ACC_ROWS, 16), _f32),
    ],
)
def _deg_kernel(dst_hbm, w_hbm, z_hbm, deg1_hbm, deg2_hbm,
                dst_v, w_v, wb_v, ones_v, acc1, acc2):
    c = lax.axis_index("c")
    s = lax.axis_index("s")
    wid = c * NS + s
    pltpu.sync_copy(dst_hbm.at[wid], dst_v)
    pltpu.sync_copy(w_hbm.at[wid], w_v)
    pltpu.sync_copy(z_hbm, acc1.at[pl.ds(s * STRIPE, STRIPE)])
    pltpu.sync_copy(z_hbm, acc2.at[pl.ds(s * STRIPE, STRIPE)])

    @pl.loop(0, BLK)
    def _(b):
        ones_v[b, :] = jnp.full((16,), 1.0, _f32)

    plsc.subcore_barrier()

    @pl.loop(0, NBLK)
    def _(j):
        @pl.loop(0, BLK, step=16)
        def _(q):
            wvec = w_v[j, pl.ds(q, 16)]
            for r in range(16):
                wb_v[q + r, :] = jnp.full((16,), 1.0, _f32) * wvec[r]

        pltpu.sync_copy(wb_v, acc1.at[dst_v.at[j]], add=True)
        pltpu.sync_copy(ones_v, acc2.at[dst_v.at[j]], add=True)

    plsc.subcore_barrier()
    sl = pl.ds(s * STRIPE, STRIPE)
    pltpu.sync_copy(acc1.at[sl], deg1_hbm.at[c, sl])
    pltpu.sync_copy(acc2.at[sl], deg2_hbm.at[c, sl])


@functools.partial(
    pl.kernel,
    mesh=_mesh,
    compiler_params=_sc_params,
    out_type=jax.ShapeDtypeStruct((NC, ACC_ROWS, D1), _f32),
    scratch_types=[
        pltpu.VMEM((NBLK, BLK), jnp.int32),
        pltpu.VMEM((NBLK, BLK), jnp.int32),
        pltpu.VMEM((NBLK, BLK), _f32),
        pltpu.VMEM((BLK, D1), _f32),
        pltpu.VMEM_SHARED((ACC_ROWS, D1), _f32),
    ],
)
def _agg1_kernel(g_hbm, src_hbm, dst_hbm, w_hbm, z_hbm, out_hbm,
                 src_v, dst_v, w_v, rbuf, acc):
    c = lax.axis_index("c")
    s = lax.axis_index("s")
    wid = c * NS + s
    pltpu.sync_copy(src_hbm.at[wid], src_v)
    pltpu.sync_copy(dst_hbm.at[wid], dst_v)
    pltpu.sync_copy(w_hbm.at[wid], w_v)
    pltpu.sync_copy(z_hbm, acc.at[pl.ds(s * STRIPE, STRIPE)])
    plsc.subcore_barrier()

    @pl.loop(0, NBLK)
    def _(j):
        pltpu.sync_copy(g_hbm.at[src_v.at[j]], rbuf)

        @pl.loop(0, BLK, step=16)
        def _(q):
            wvec = w_v[j, pl.ds(q, 16)]
            for r in range(16):
                wb = wvec[r]
                for cc in range(D1 // 16):
                    csl = pl.ds(cc * 16, 16)
                    rbuf[q + r, csl] = rbuf[q + r, csl] * wb

        pltpu.sync_copy(rbuf, acc.at[dst_v.at[j]], add=True)

    plsc.subcore_barrier()
    sl = pl.ds(s * STRIPE, STRIPE)
    pltpu.sync_copy(acc.at[sl], out_hbm.at[c, sl])


@functools.partial(
    pl.kernel,
    mesh=_mesh,
    compiler_params=_sc_params,
    out_type=jax.ShapeDtypeStruct((NC, ACC_ROWS, D2P), _f32),
    scratch_types=[
        pltpu.VMEM((NBLK, BLK), jnp.int32),
        pltpu.VMEM((NBLK, BLK), jnp.int32),
        pltpu.VMEM((BLK, D2P), _f32),
        pltpu.VMEM_SHARED((ACC_ROWS, D2P), _f32),
    ],
)
def _agg2_kernel(g_hbm, src_hbm, dst_hbm, z_hbm, out_hbm,
                 src_v, dst_v, rbuf, acc):
    c = lax.axis_index("c")
    s = lax.axis_index("s")
    wid = c * NS + s
    pltpu.sync_copy(src_hbm.at[wid], src_v)
    pltpu.sync_copy(dst_hbm.at[wid], dst_v)
    pltpu.sync_copy(z_hbm, acc.at[pl.ds(s * STRIPE, STRIPE)])
    plsc.subcore_barrier()

    @pl.loop(0, NBLK)
    def _(j):
        pltpu.sync_copy(g_hbm.at[src_v.at[j]], rbuf)
        pltpu.sync_copy(rbuf, acc.at[dst_v.at[j]], add=True)

    plsc.subcore_barrier()
    sl = pl.ds(s * STRIPE, STRIPE)
    pltpu.sync_copy(acc.at[sl], out_hbm.at[c, sl])


RB = 2000
_G = N // RB


def _row_spec(d):
    return pl.BlockSpec((RB, d), lambda i: (i, 0))


def _acc_spec(d):
    return pl.BlockSpec((NC, RB, d), lambda i: (0, i, 0))


def _full_spec(r, d):
    return pl.BlockSpec((r, d), lambda i: (0, 0))


def _dinv(deg_ref, extra):
    d = deg_ref[0, :, 0:1] + deg_ref[1, :, 0:1] + extra
    return jnp.where(d > 0, lax.rsqrt(d), 0.0)


def _b_body(x_ref, w1_ref, deg1_ref, o_ref):
    h = jnp.dot(x_ref[...], w1_ref[...], precision=lax.Precision.HIGHEST,
                preferred_element_type=_f32)
    o_ref[...] = h * _dinv(deg1_ref, 1.0)


def _d_body(acc1_ref, g1_ref, deg1_ref, deg2_ref, w2_ref, b1_ref, o_ref):
    ssum = acc1_ref[0] + acc1_ref[1] + g1_ref[...]
    out1 = jnp.maximum(ssum * _dinv(deg1_ref, 1.0) + b1_ref[...], 0.0)
    h2 = jnp.dot(out1, w2_ref[...], precision=lax.Precision.HIGHEST,
                 preferred_element_type=_f32)
    o_ref[...] = h2 * _dinv(deg2_ref, 1.0)


def _f_body(acc2_ref, g2_ref, deg2_ref, b2_ref, o_ref):
    z = (acc2_ref[0] + acc2_ref[1] + g2_ref[...]) \
        * _dinv(deg2_ref, 1.0) + b2_ref[...]
    col = lax.broadcasted_iota(jnp.int32, (RB, D2P), 1)
    valid = col < D2
    m = jnp.max(jnp.where(valid, z, -jnp.inf), axis=1, keepdims=True)
    ex = jnp.where(valid, jnp.exp(z - m), 0.0)
    lse = jnp.log(jnp.sum(ex, axis=1, keepdims=True))
    o_ref[...] = (z - m - lse)[:, :D2]


def _tc_call(body, in_specs, out_d, out_shape=None):
    return pl.pallas_call(
        body,
        grid=(_G,),
        in_specs=in_specs,
        out_specs=_row_spec(out_d),
        out_shape=jax.ShapeDtypeStruct(out_shape or (N, out_d), _f32),
    )


def kernel(x, adj_t, edge_weight, W1, b1, W2, b2):
    src = adj_t[0].astype(jnp.int32)
    dst = adj_t[1].astype(jnp.int32)
    w = edge_weight.astype(_f32)

    pad = EP - E
    src3 = jnp.concatenate([src, jnp.zeros((pad,), jnp.int32)]).reshape(NW, NBLK, BLK)
    dst3 = jnp.concatenate([dst, jnp.full((pad,), N, jnp.int32)]).reshape(NW, NBLK, BLK)
    w3 = jnp.concatenate([w, jnp.zeros((pad,), _f32)]).reshape(NW, NBLK, BLK)

    z16 = jnp.zeros((STRIPE, 16), _f32)
    z128 = jnp.zeros((STRIPE, D1), _f32)
    z48 = jnp.zeros((STRIPE, D2P), _f32)

    w2p = jnp.concatenate([W2, jnp.zeros((D1, D2P - D2), _f32)], axis=1)
    b1r = b1.reshape(1, D1)
    b2p = jnp.concatenate([b2, jnp.zeros((D2P - D2,), _f32)]).reshape(1, D2P)

    deg1p, deg2p = _deg_kernel(dst3, w3, z16)
    g1 = _tc_call(
        _b_body, [_row_spec(D1), _full_spec(D1, D1), _acc_spec(16)], D1
    )(x, W1, deg1p)
    acc1 = _agg1_kernel(g1, src3, dst3, w3, z128)
    g2 = _tc_call(
        _d_body,
        [_acc_spec(D1), _row_spec(D1), _acc_spec(16), _acc_spec(16),
         _full_spec(D1, D2P), _full_spec(1, D1)],
        D2P,
    )(acc1, g1, deg1p, deg2p, w2p, b1r)
    acc2 = _agg2_kernel(g2, src3, dst3, z48)
    return _tc_call(
        _f_body,
        [_acc_spec(D2P), _row_spec(D2P), _acc_spec(16), _full_spec(1, D2P)],
        D2,
    )(acc2, g2, deg2p, b2p)


# speedup vs baseline: 16.3171x; 1.1286x over previous
import functools

import jax
import jax.numpy as jnp
from jax import lax
from jax.experimental import pallas as pl
from jax.experimental.pallas import tpu as pltpu
from jax.experimental.pallas import tpu_sc as plsc

N = 10000
E = 320000
D1 = 128
D2 = 40
D2P = 48

NC = 2
NS = 16
NW = NC * NS
BLK = 128
NBLK = 80
EP = NW * NBLK * BLK
NBLK1 = 159
EP1 = NS * NBLK1 * BLK
ACC_ROWS = 10240
STRIPE = ACC_ROWS // NS

_mesh = plsc.VectorSubcoreMesh(core_axis_name="c", subcore_axis_name="s")
_sc_params = pltpu.CompilerParams(use_tc_tiling_on_sc=False)
_f32 = jnp.float32


@functools.partial(
    pl.kernel,
    mesh=_mesh,
    compiler_params=_sc_params,
    out_type=(
        jax.ShapeDtypeStruct((NC, ACC_ROWS, 16), _f32),
        jax.ShapeDtypeStruct((NC, ACC_ROWS, 16), _f32),
    ),
    scratch_types=[
        pltpu.VMEM((NBLK, BLK), jnp.int32),
        pltpu.VMEM((NBLK, BLK), _f32),
        pltpu.VMEM((BLK, 16), _f32),
        pltpu.VMEM((BLK, 16), _f32),
        pltpu.VMEM_SHARED((ACC_ROWS, 16), _f32),
        pltpu.VMEM_SHARED((ACC_ROWS, 16), _f32),
    ],
)
def _deg_kernel(dst_hbm, w_hbm, z_hbm, deg1_hbm, deg2_hbm,
                dst_v, w_v, wb_v, ones_v, acc1, acc2):
    c = lax.axis_index("c")
    s = lax.axis_index("s")
    wid = c * NS + s
    pltpu.sync_copy(dst_hbm.at[wid], dst_v)
    pltpu.sync_copy(w_hbm.at[wid], w_v)
    pltpu.sync_copy(z_hbm, acc1.at[pl.ds(s * STRIPE, STRIPE)])
    pltpu.sync_copy(z_hbm, acc2.at[pl.ds(s * STRIPE, STRIPE)])

    @pl.loop(0, BLK)
    def _(b):
        ones_v[b, :] = jnp.full((16,), 1.0, _f32)

    plsc.subcore_barrier()

    @pl.loop(0, NBLK)
    def _(j):
        @pl.loop(0, BLK, step=16)
        def _(q):
            wvec = w_v[j, pl.ds(q, 16)]
            for r in range(16):
                wb_v[q + r, :] = jnp.full((16,), 1.0, _f32) * wvec[r]

        pltpu.sync_copy(wb_v, acc1.at[dst_v.at[j]], add=True)
        pltpu.sync_copy(ones_v, acc2.at[dst_v.at[j]], add=True)

    plsc.subcore_barrier()
    sl = pl.ds(s * STRIPE, STRIPE)
    pltpu.sync_copy(acc1.at[sl], deg1_hbm.at[c, sl])
    pltpu.sync_copy(acc2.at[sl], deg2_hbm.at[c, sl])


HD1 = D1 // 2


@functools.partial(
    pl.kernel,
    mesh=_mesh,
    compiler_params=_sc_params,
    out_type=jax.ShapeDtypeStruct((ACC_ROWS, NC, HD1), _f32),
    scratch_types=[
        pltpu.VMEM((NBLK1, BLK), jnp.int32),
        pltpu.VMEM((NBLK1, BLK), jnp.int32),
        pltpu.VMEM((NBLK1, BLK), _f32),
        pltpu.VMEM((BLK, HD1), _f32),
        pltpu.VMEM((BLK, HD1), _f32),
        pltpu.VMEM((BLK, HD1), _f32),
        pltpu.VMEM_SHARED((ACC_ROWS, HD1), _f32),
        pltpu.SemaphoreType.DMA,
        pltpu.SemaphoreType.DMA,
        pltpu.SemaphoreType.DMA,
        pltpu.SemaphoreType.DMA,
        pltpu.SemaphoreType.DMA,
        pltpu.SemaphoreType.DMA,
    ],
)
def _agg1_kernel(g_hbm, src_hbm, dst_hbm, w_hbm, z_hbm, out_hbm,
                 src_v, dst_v, w_v, b0, b1, b2,
                 acc, gs0, gs1, gs2, ss0, ss1, ss2):
    c = lax.axis_index("c")
    s = lax.axis_index("s")
    pltpu.sync_copy(src_hbm.at[s], src_v)
    pltpu.sync_copy(dst_hbm.at[s], dst_v)
    pltpu.sync_copy(w_hbm.at[s], w_v)
    pltpu.sync_copy(z_hbm, acc.at[pl.ds(s * STRIPE, STRIPE)])

    @pl.loop(0, NBLK1)
    def _(jr):
        for cc in range(BLK // 16):
            csl = pl.ds(cc * 16, 16)
            src_v[jr, csl] = src_v[jr, csl] * 2 + c

    plsc.subcore_barrier()

    bufs = [b0, b1, b2]
    gsems = [gs0, gs1, gs2]
    ssems = [ss0, ss1, ss2]

    def gather_start(j, k):
        pltpu.async_copy(g_hbm.at[src_v.at[j]], bufs[k], gsems[k])

    def gather_wait(j, k):
        pltpu.make_async_copy(g_hbm.at[src_v.at[j]], bufs[k], gsems[k]).wait()

    def scatter_start(j, k):
        pltpu.async_copy(bufs[k], acc.at[dst_v.at[j]], ssems[k], add=True)

    def scatter_wait(j, k):
        pltpu.make_async_copy(bufs[k], acc.at[dst_v.at[j]], ssems[k]).wait()

    def multiply(j, k):
        @pl.loop(0, BLK, step=16)
        def _(q):
            wvec = w_v[j, pl.ds(q, 16)]
            for r in range(16):
                wb = wvec[r]
                for cc in range(HD1 // 16):
                    csl = pl.ds(cc * 16, 16)
                    bufs[k][q + r, csl] = bufs[k][q + r, csl] * wb

    gather_start(0, 0)
    gather_start(1, 1)

    gather_wait(0, 0)
    gather_start(2, 2)
    multiply(0, 0)
    scatter_start(0, 0)

    gather_wait(1, 1)
    multiply(1, 1)
    scatter_start(1, 1)

    @pl.loop(2, NBLK1 - 1, step=3)
    def _(base):
        for kk in range(3):
            j = base + kk
            k = (2 + kk) % 3
            k2 = (k + 1) % 3
            gather_wait(j, k)
            scatter_wait(j - 2, k2)
            gather_start(j + 1, k2)
            multiply(j, k)
            scatter_start(j, k)

    jl = NBLK1 - 1
    kl = jl % 3
    gather_wait(jl, kl)
    scatter_wait(jl - 2, (jl + 1) % 3)
    multiply(jl, kl)
    scatter_start(jl, kl)
    scatter_wait(jl - 1, (jl - 1) % 3)
    scatter_wait(jl, kl)

    plsc.subcore_barrier()
    sl = pl.ds(s * STRIPE, STRIPE)
    pltpu.sync_copy(acc.at[sl], out_hbm.at[sl, c])


@functools.partial(
    pl.kernel,
    mesh=_mesh,
    compiler_params=_sc_params,
    out_type=jax.ShapeDtypeStruct((NC, ACC_ROWS, D2P), _f32),
    scratch_types=[
        pltpu.VMEM((NBLK, BLK), jnp.int32),
        pltpu.VMEM((NBLK, BLK), jnp.int32),
        pltpu.VMEM((BLK, D2P), _f32),
        pltpu.VMEM((BLK, D2P), _f32),
        pltpu.VMEM((BLK, D2P), _f32),
        pltpu.VMEM((BLK, D2P), _f32),
        pltpu.VMEM_SHARED((ACC_ROWS, D2P), _f32),
        pltpu.SemaphoreType.DMA,
        pltpu.SemaphoreType.DMA,
        pltpu.SemaphoreType.DMA,
        pltpu.SemaphoreType.DMA,
        pltpu.SemaphoreType.DMA,
        pltpu.SemaphoreType.DMA,
        pltpu.SemaphoreType.DMA,
        pltpu.SemaphoreType.DMA,
    ],
)
def _agg2_kernel(g_hbm, src_hbm, dst_hbm, z_hbm, out_hbm,
                 src_v, dst_v, b0, b1, b2, b3,
                 acc, gs0, gs1, gs2, gs3, ss0, ss1, ss2, ss3):
    c = lax.axis_index("c")
    s = lax.axis_index("s")
    wid = c * NS + s
    pltpu.sync_copy(src_hbm.at[wid], src_v)
    pltpu.sync_copy(dst_hbm.at[wid], dst_v)
    pltpu.sync_copy(z_hbm, acc.at[pl.ds(s * STRIPE, STRIPE)])
    plsc.subcore_barrier()

    bufs = [b0, b1, b2, b3]
    gsems = [gs0, gs1, gs2, gs3]
    ssems = [ss0, ss1, ss2, ss3]

    def gather_start(j, k):
        pltpu.async_copy(g_hbm.at[src_v.at[j]], bufs[k], gsems[k])

    def gather_wait(j, k):
        pltpu.make_async_copy(g_hbm.at[src_v.at[j]], bufs[k], gsems[k]).wait()

    def scatter_start(j, k):
        pltpu.async_copy(bufs[k], acc.at[dst_v.at[j]], ssems[k], add=True)

    def scatter_wait(j, k):
        pltpu.make_async_copy(bufs[k], acc.at[dst_v.at[j]], ssems[k]).wait()

    gather_start(0, 0)
    gather_start(1, 1)
    for j in (0, 1):
        gather_wait(j, j)
        scatter_start(j, j)
        gather_start(j + 2, j + 2)

    @pl.loop(2, NBLK - 2, step=4)
    def _(base):
        for kk in range(4):
            j = base + kk
            k = (2 + kk) % 4
            k2 = (k + 2) % 4
            gather_wait(j, k)
            scatter_start(j, k)
            scatter_wait(j - 2, k2)
            gather_start(j + 2, k2)

    for j in (NBLK - 2, NBLK - 1):
        k = j % 4
        gather_wait(j, k)
        scatter_start(j, k)
        scatter_wait(j - 2, (k + 2) % 4)
    scatter_wait(NBLK - 2, (NBLK - 2) % 4)
    scatter_wait(NBLK - 1, (NBLK - 1) % 4)

    plsc.subcore_barrier()
    sl = pl.ds(s * STRIPE, STRIPE)
    pltpu.sync_copy(acc.at[sl], out_hbm.at[c, sl])


RB = 2000
_G = N // RB


def _row_spec(d):
    return pl.BlockSpec((RB, d), lambda i: (i, 0))


def _acc_spec(d):
    return pl.BlockSpec((NC, RB, d), lambda i: (0, i, 0))


def _full_spec(r, d):
    return pl.BlockSpec((r, d), lambda i: (0, 0))


def _dinv(deg_ref, extra):
    d = deg_ref[0, :, 0:1] + deg_ref[1, :, 0:1] + extra
    return jnp.where(d > 0, lax.rsqrt(d), 0.0)


def _b_body(x_ref, w1_ref, deg1_ref, o_ref):
    h = jnp.dot(x_ref[...], w1_ref[...], precision=lax.Precision.HIGHEST,
                preferred_element_type=_f32)
    o_ref[...] = h * _dinv(deg1_ref, 1.0)


def _d_body(acc1_ref, g1_ref, deg1_ref, deg2_ref, w2_ref, b1_ref, o_ref):
    ssum = acc1_ref[...] + g1_ref[...]
    out1 = jnp.maximum(ssum * _dinv(deg1_ref, 1.0) + b1_ref[...], 0.0)
    h2 = jnp.dot(out1, w2_ref[...], precision=lax.Precision.HIGHEST,
                 preferred_element_type=_f32)
    o_ref[...] = h2 * _dinv(deg2_ref, 1.0)


def _f_body(acc2_ref, g2_ref, deg2_ref, b2_ref, o_ref):
    z = (acc2_ref[0] + acc2_ref[1] + g2_ref[...]) \
        * _dinv(deg2_ref, 1.0) + b2_ref[...]
    col = lax.broadcasted_iota(jnp.int32, (RB, D2P), 1)
    valid = col < D2
    m = jnp.max(jnp.where(valid, z, -jnp.inf), axis=1, keepdims=True)
    ex = jnp.where(valid, jnp.exp(z - m), 0.0)
    lse = jnp.log(jnp.sum(ex, axis=1, keepdims=True))
    o_ref[...] = (z - m - lse)[:, :D2]


def _tc_call(body, in_specs, out_d, out_shape=None):
    return pl.pallas_call(
        body,
        grid=(_G,),
        in_specs=in_specs,
        out_specs=_row_spec(out_d),
        out_shape=jax.ShapeDtypeStruct(out_shape or (N, out_d), _f32),
    )


def kernel(x, adj_t, edge_weight, W1, b1, W2, b2):
    src = adj_t[0].astype(jnp.int32)
    dst = adj_t[1].astype(jnp.int32)
    w = edge_weight.astype(_f32)

    pad = EP - E
    src3 = jnp.concatenate([src, jnp.zeros((pad,), jnp.int32)]).reshape(NW, NBLK, BLK)
    dst3 = jnp.concatenate([dst, jnp.full((pad,), N, jnp.int32)]).reshape(NW, NBLK, BLK)
    w3 = jnp.concatenate([w, jnp.zeros((pad,), _f32)]).reshape(NW, NBLK, BLK)

    pad1 = EP1 - E
    src1 = jnp.concatenate([src, jnp.zeros((pad1,), jnp.int32)]).reshape(NS, NBLK1, BLK)
    dst1 = jnp.concatenate([dst, jnp.full((pad1,), N, jnp.int32)]).reshape(NS, NBLK1, BLK)
    w1 = jnp.concatenate([w, jnp.zeros((pad1,), _f32)]).reshape(NS, NBLK1, BLK)

    z16 = jnp.zeros((STRIPE, 16), _f32)
    z64 = jnp.zeros((STRIPE, HD1), _f32)
    z48 = jnp.zeros((STRIPE, D2P), _f32)

    w2p = jnp.concatenate([W2, jnp.zeros((D1, D2P - D2), _f32)], axis=1)
    b1r = b1.reshape(1, D1)
    b2p = jnp.concatenate([b2, jnp.zeros((D2P - D2,), _f32)]).reshape(1, D2P)

    deg1p, deg2p = _deg_kernel(dst3, w3, z16)
    g1 = _tc_call(
        _b_body, [_row_spec(D1), _full_spec(D1, D1), _acc_spec(16)], D1
    )(x, W1, deg1p)
    acc1 = _agg1_kernel(g1.reshape(2 * N, HD1), src1, dst1, w1, z64)
    g2 = _tc_call(
        _d_body,
        [_row_spec(D1), _row_spec(D1), _acc_spec(16), _acc_spec(16),
         _full_spec(D1, D2P), _full_spec(1, D1)],
        D2P,
    )(acc1.reshape(ACC_ROWS, D1), g1, deg1p, deg2p, w2p, b1r)
    acc2 = _agg2_kernel(g2, src3, dst3, z48)
    return _tc_call(
        _f_body,
        [_acc_spec(D2P), _row_spec(D2P), _acc_spec(16), _full_spec(1, D2P)],
        D2,
    )(acc2, g2, deg2p, b2p)


# speedup vs baseline: 16.6140x; 1.0182x over previous
import functools

import jax
import jax.numpy as jnp
from jax import lax
from jax.experimental import pallas as pl
from jax.experimental.pallas import tpu as pltpu
from jax.experimental.pallas import tpu_sc as plsc

N = 10000
E = 320000
D1 = 128
D2 = 40
D2P = 48

NC = 2
NS = 16
NW = NC * NS
BLK = 128
NBLK = 80
EP = NW * NBLK * BLK
ACC_ROWS = 10240
STRIPE = ACC_ROWS // NS

_mesh = plsc.VectorSubcoreMesh(core_axis_name="c", subcore_axis_name="s")
_sc_params = pltpu.CompilerParams(use_tc_tiling_on_sc=False)
_f32 = jnp.float32


@functools.partial(
    pl.kernel,
    mesh=_mesh,
    compiler_params=_sc_params,
    out_type=(
        jax.ShapeDtypeStruct((NC, ACC_ROWS, 16), _f32),
        jax.ShapeDtypeStruct((NC, ACC_ROWS, 16), _f32),
    ),
    scratch_types=[
        pltpu.VMEM((NBLK, BLK), jnp.int32),
        pltpu.VMEM((NBLK, BLK), _f32),
        pltpu.VMEM((BLK, 16), _f32),
        pltpu.VMEM((BLK, 16), _f32),
        pltpu.VMEM_SHARED((ACC_ROWS, 16), _f32),
        pltpu.VMEM_SHARED((ACC_ROWS, 16), _f32),
    ],
)
def _deg_kernel(dst_hbm, w_hbm, z_hbm, deg1_hbm, deg2_hbm,
                dst_v, w_v, wb_v, ones_v, acc1, acc2):
    c = lax.axis_index("c")
    s = lax.axis_index("s")
    wid = c * NS + s
    pltpu.sync_copy(dst_hbm.at[wid], dst_v)
    pltpu.sync_copy(w_hbm.at[wid], w_v)
    pltpu.sync_copy(z_hbm, acc1.at[pl.ds(s * STRIPE, STRIPE)])
    pltpu.sync_copy(z_hbm, acc2.at[pl.ds(s * STRIPE, STRIPE)])

    @pl.loop(0, BLK)
    def _(b):
        ones_v[b, :] = jnp.full((16,), 1.0, _f32)

    plsc.subcore_barrier()

    @pl.loop(0, NBLK)
    def _(j):
        @pl.loop(0, BLK, step=16)
        def _(q):
            wvec = w_v[j, pl.ds(q, 16)]
            for r in range(16):
                wb_v[q + r, :] = jnp.full((16,), 1.0, _f32) * wvec[r]

        pltpu.sync_copy(wb_v, acc1.at[dst_v.at[j]], add=True)
        pltpu.sync_copy(ones_v, acc2.at[dst_v.at[j]], add=True)

    plsc.subcore_barrier()
    sl = pl.ds(s * STRIPE, STRIPE)
    pltpu.sync_copy(acc1.at[sl], deg1_hbm.at[c, sl])
    pltpu.sync_copy(acc2.at[sl], deg2_hbm.at[c, sl])


CB = 8
NCH = NBLK // CB


@functools.partial(
    pl.kernel,
    mesh=_mesh,
    compiler_params=_sc_params,
    out_type=jax.ShapeDtypeStruct((NC, ACC_ROWS, D1), _f32),
    scratch_types=[
        pltpu.VMEM((2 * CB, BLK), jnp.int32),
        pltpu.VMEM((2 * CB, BLK), jnp.int32),
        pltpu.VMEM((2 * CB, BLK), _f32),
        pltpu.VMEM((BLK, D1), _f32),
        pltpu.VMEM((BLK, D1), _f32),
        pltpu.VMEM_SHARED((ACC_ROWS, D1), _f32),
        pltpu.SemaphoreType.DMA,
        pltpu.SemaphoreType.DMA,
        pltpu.SemaphoreType.DMA,
        pltpu.SemaphoreType.DMA,
        pltpu.SemaphoreType.DMA,
    ],
)
def _agg1_kernel(g_hbm, src_hbm, dst_hbm, w_hbm, z_hbm, out_hbm,
                 srcC, dstC, wC, b0, b1,
                 acc, gs0, gs1, ss0, ss1, csem):
    c = lax.axis_index("c")
    s = lax.axis_index("s")
    wid = c * NS + s
    pltpu.sync_copy(z_hbm, acc.at[pl.ds(s * STRIPE, STRIPE)])

    bufs = [b0, b1]
    gsems = [gs0, gs1]
    ssems = [ss0, ss1]

    def crow(j):
        return ((j // CB) % 2) * CB + (j % CB)

    def stage_start(ch):
        sl = pl.ds(ch * CB, CB)
        half = pl.ds((ch % 2) * CB, CB)
        pltpu.async_copy(src_hbm.at[wid, sl], srcC.at[half], csem)
        pltpu.async_copy(dst_hbm.at[wid, sl], dstC.at[half], csem)
        pltpu.async_copy(w_hbm.at[wid, sl], wC.at[half], csem)

    def stage_wait(ch):
        sl = pl.ds(ch * CB, CB)
        half = pl.ds((ch % 2) * CB, CB)
        pltpu.make_async_copy(src_hbm.at[wid, sl], srcC.at[half], csem).wait()
        pltpu.make_async_copy(dst_hbm.at[wid, sl], dstC.at[half], csem).wait()
        pltpu.make_async_copy(w_hbm.at[wid, sl], wC.at[half], csem).wait()

    def gather_start(j, k):
        pltpu.async_copy(g_hbm.at[srcC.at[crow(j)]], bufs[k], gsems[k])

    def gather_wait(j, k):
        pltpu.make_async_copy(g_hbm.at[srcC.at[crow(j)]], bufs[k],
                              gsems[k]).wait()

    def scatter_start(j, k):
        pltpu.async_copy(bufs[k], acc.at[dstC.at[crow(j)]], ssems[k], add=True)

    def scatter_wait(j, k):
        pltpu.make_async_copy(bufs[k], acc.at[dstC.at[crow(j)]],
                              ssems[k]).wait()

    def multiply(j, k):
        rw = crow(j)

        @pl.loop(0, BLK, step=16)
        def _(q):
            wvec = wC[rw, pl.ds(q, 16)]
            for r in range(16):
                wb = wvec[r]
                for cc in range(D1 // 16):
                    csl = pl.ds(cc * 16, 16)
                    bufs[k][q + r, csl] = bufs[k][q + r, csl] * wb

    def manage_chunks(j):
        ch = j // CB

        @pl.when(jnp.logical_and(j % CB == 1,
                                 jnp.logical_and(j >= CB + 1, ch + 1 < NCH)))
        def _():
            stage_start(ch + 1)

        @pl.when(jnp.logical_and(j % CB == CB - 2, ch + 1 < NCH))
        def _():
            stage_wait(ch + 1)

    stage_start(0)
    stage_start(1)
    stage_wait(0)
    plsc.subcore_barrier()

    gather_start(0, 0)
    gather_wait(0, 0)
    gather_start(1, 1)
    multiply(0, 0)
    scatter_start(0, 0)

    @pl.loop(1, NBLK - 3, step=2)
    def _(base):
        for kk in range(2):
            j = base + kk
            k = (1 + kk) % 2
            k2 = (k + 1) % 2
            gather_wait(j, k)
            scatter_wait(j - 1, k2)
            gather_start(j + 1, k2)
            manage_chunks(j)
            multiply(j, k)
            scatter_start(j, k)

    for j in (NBLK - 3, NBLK - 2):
        k = j % 2
        k2 = (k + 1) % 2
        gather_wait(j, k)
        scatter_wait(j - 1, k2)
        gather_start(j + 1, k2)
        multiply(j, k)
        scatter_start(j, k)

    jl = NBLK - 1
    kl = jl % 2
    gather_wait(jl, kl)
    scatter_wait(jl - 1, (kl + 1) % 2)
    multiply(jl, kl)
    scatter_start(jl, kl)
    scatter_wait(jl, kl)

    plsc.subcore_barrier()
    sl = pl.ds(s * STRIPE, STRIPE)
    pltpu.sync_copy(acc.at[sl], out_hbm.at[c, sl])


@functools.partial(
    pl.kernel,
    mesh=_mesh,
    compiler_params=_sc_params,
    out_type=jax.ShapeDtypeStruct((NC, ACC_ROWS, D2P), _f32),
    scratch_types=[
        pltpu.VMEM((NBLK, BLK), jnp.int32),
        pltpu.VMEM((NBLK, BLK), jnp.int32),
        pltpu.VMEM((BLK, D2P), _f32),
        pltpu.VMEM((BLK, D2P), _f32),
        pltpu.VMEM((BLK, D2P), _f32),
        pltpu.VMEM((BLK, D2P), _f32),
        pltpu.VMEM_SHARED((ACC_ROWS, D2P), _f32),
        pltpu.SemaphoreType.DMA,
        pltpu.SemaphoreType.DMA,
        pltpu.SemaphoreType.DMA,
        pltpu.SemaphoreType.DMA,
        pltpu.SemaphoreType.DMA,
        pltpu.SemaphoreType.DMA,
        pltpu.SemaphoreType.DMA,
        pltpu.SemaphoreType.DMA,
    ],
)
def _agg2_kernel(g_hbm, src_hbm, dst_hbm, z_hbm, out_hbm,
                 src_v, dst_v, b0, b1, b2, b3,
                 acc, gs0, gs1, gs2, gs3, ss0, ss1, ss2, ss3):
    c = lax.axis_index("c")
    s = lax.axis_index("s")
    wid = c * NS + s
    pltpu.sync_copy(src_hbm.at[wid], src_v)
    pltpu.sync_copy(dst_hbm.at[wid], dst_v)
    pltpu.sync_copy(z_hbm, acc.at[pl.ds(s * STRIPE, STRIPE)])
    plsc.subcore_barrier()

    bufs = [b0, b1, b2, b3]
    gsems = [gs0, gs1, gs2, gs3]
    ssems = [ss0, ss1, ss2, ss3]

    def gather_start(j, k):
        pltpu.async_copy(g_hbm.at[src_v.at[j]], bufs[k], gsems[k])

    def gather_wait(j, k):
        pltpu.make_async_copy(g_hbm.at[src_v.at[j]], bufs[k], gsems[k]).wait()

    def scatter_start(j, k):
        pltpu.async_copy(bufs[k], acc.at[dst_v.at[j]], ssems[k], add=True)

    def scatter_wait(j, k):
        pltpu.make_async_copy(bufs[k], acc.at[dst_v.at[j]], ssems[k]).wait()

    gather_start(0, 0)
    gather_start(1, 1)
    for j in (0, 1):
        gather_wait(j, j)
        scatter_start(j, j)
        gather_start(j + 2, j + 2)

    @pl.loop(2, NBLK - 2, step=4)
    def _(base):
        for kk in range(4):
            j = base + kk
            k = (2 + kk) % 4
            k2 = (k + 2) % 4
            gather_wait(j, k)
            scatter_start(j, k)
            scatter_wait(j - 2, k2)
            gather_start(j + 2, k2)

    for j in (NBLK - 2, NBLK - 1):
        k = j % 4
        gather_wait(j, k)
        scatter_start(j, k)
        scatter_wait(j - 2, (k + 2) % 4)
    scatter_wait(NBLK - 2, (NBLK - 2) % 4)
    scatter_wait(NBLK - 1, (NBLK - 1) % 4)

    plsc.subcore_barrier()
    sl = pl.ds(s * STRIPE, STRIPE)
    pltpu.sync_copy(acc.at[sl], out_hbm.at[c, sl])


RB = 2000
_G = N // RB


def _row_spec(d):
    return pl.BlockSpec((RB, d), lambda i: (i, 0))


def _acc_spec(d):
    return pl.BlockSpec((NC, RB, d), lambda i: (0, i, 0))


def _full_spec(r, d):
    return pl.BlockSpec((r, d), lambda i: (0, 0))


def _dinv(deg_ref, extra):
    d = deg_ref[0, :, 0:1] + deg_ref[1, :, 0:1] + extra
    return jnp.where(d > 0, lax.rsqrt(d), 0.0)


def _b_body(x_ref, w1_ref, deg1_ref, o_ref):
    h = jnp.dot(x_ref[...], w1_ref[...], precision=lax.Precision.HIGHEST,
                preferred_element_type=_f32)
    o_ref[...] = h * _dinv(deg1_ref, 1.0)


def _d_body(acc1_ref, g1_ref, deg1_ref, deg2_ref, w2_ref, b1_ref, o_ref):
    ssum = acc1_ref[0] + acc1_ref[1] + g1_ref[...]
    out1 = jnp.maximum(ssum * _dinv(deg1_ref, 1.0) + b1_ref[...], 0.0)
    h2 = jnp.dot(out1, w2_ref[...], precision=lax.Precision.HIGHEST,
                 preferred_element_type=_f32)
    o_ref[...] = h2 * _dinv(deg2_ref, 1.0)


def _f_body(acc2_ref, g2_ref, deg2_ref, b2_ref, o_ref):
    z = (acc2_ref[0] + acc2_ref[1] + g2_ref[...]) \
        * _dinv(deg2_ref, 1.0) + b2_ref[...]
    col = lax.broadcasted_iota(jnp.int32, (RB, D2P), 1)
    valid = col < D2
    m = jnp.max(jnp.where(valid, z, -jnp.inf), axis=1, keepdims=True)
    ex = jnp.where(valid, jnp.exp(z - m), 0.0)
    lse = jnp.log(jnp.sum(ex, axis=1, keepdims=True))
    o_ref[...] = (z - m - lse)[:, :D2]


def _tc_call(body, in_specs, out_d, out_shape=None):
    return pl.pallas_call(
        body,
        grid=(_G,),
        in_specs=in_specs,
        out_specs=_row_spec(out_d),
        out_shape=jax.ShapeDtypeStruct(out_shape or (N, out_d), _f32),
    )


def kernel(x, adj_t, edge_weight, W1, b1, W2, b2):
    src = adj_t[0].astype(jnp.int32)
    dst = adj_t[1].astype(jnp.int32)
    w = edge_weight.astype(_f32)

    pad = EP - E
    src3 = jnp.concatenate([src, jnp.zeros((pad,), jnp.int32)]).reshape(NW, NBLK, BLK)
    dst3 = jnp.concatenate([dst, jnp.full((pad,), N, jnp.int32)]).reshape(NW, NBLK, BLK)
    w3 = jnp.concatenate([w, jnp.zeros((pad,), _f32)]).reshape(NW, NBLK, BLK)

    z16 = jnp.zeros((STRIPE, 16), _f32)
    z128 = jnp.zeros((STRIPE, D1), _f32)
    z48 = jnp.zeros((STRIPE, D2P), _f32)

    w2p = jnp.concatenate([W2, jnp.zeros((D1, D2P - D2), _f32)], axis=1)
    b1r = b1.reshape(1, D1)
    b2p = jnp.concatenate([b2, jnp.zeros((D2P - D2,), _f32)]).reshape(1, D2P)

    deg1p, deg2p = _deg_kernel(dst3, w3, z16)
    g1 = _tc_call(
        _b_body, [_row_spec(D1), _full_spec(D1, D1), _acc_spec(16)], D1
    )(x, W1, deg1p)
    acc1 = _agg1_kernel(g1, src3, dst3, w3, z128)
    g2 = _tc_call(
        _d_body,
        [_acc_spec(D1), _row_spec(D1), _acc_spec(16), _acc_spec(16),
         _full_spec(D1, D2P), _full_spec(1, D1)],
        D2P,
    )(acc1, g1, deg1p, deg2p, w2p, b1r)
    acc2 = _agg2_kernel(g2, src3, dst3, z48)
    return _tc_call(
        _f_body,
        [_acc_spec(D2P), _row_spec(D2P), _acc_spec(16), _full_spec(1, D2P)],
        D2,
    )(acc2, g2, deg2p, b2p)


# speedup vs baseline: 19.3004x; 1.1617x over previous
import functools

import numpy as np

import jax
import jax.numpy as jnp
from jax import lax
from jax.experimental import pallas as pl
from jax.experimental.pallas import tpu as pltpu
from jax.experimental.pallas import tpu_sc as plsc

N = 10000
E = 320000
D1 = 128
D2 = 40
D2P = 48

NC = 2
NS = 16
NW = NC * NS
BLK = 128
NBLK = 80
EP = NW * NBLK * BLK
ACC_ROWS = 10240
STRIPE = ACC_ROWS // NS

_mesh = plsc.VectorSubcoreMesh(core_axis_name="c", subcore_axis_name="s")
_sc_params = pltpu.CompilerParams(use_tc_tiling_on_sc=False)
_sc_params_nl = pltpu.CompilerParams(use_tc_tiling_on_sc=False,
                                     needs_layout_passes=False)
_f32 = jnp.float32


@functools.partial(
    pl.kernel,
    mesh=_mesh,
    compiler_params=_sc_params,
    out_type=(
        jax.ShapeDtypeStruct((NC, ACC_ROWS, 16), _f32),
        jax.ShapeDtypeStruct((NC, ACC_ROWS, 16), _f32),
    ),
    scratch_types=[
        pltpu.VMEM((NBLK, BLK), jnp.int32),
        pltpu.VMEM((NBLK, BLK), _f32),
        pltpu.VMEM((BLK, 16), _f32),
        pltpu.VMEM((BLK, 16), _f32),
        pltpu.VMEM_SHARED((ACC_ROWS, 16), _f32),
        pltpu.VMEM_SHARED((ACC_ROWS, 16), _f32),
    ],
)
def _deg_kernel(dst_hbm, w_hbm, z_hbm, deg1_hbm, deg2_hbm,
                dst_v, w_v, wb_v, ones_v, acc1, acc2):
    c = lax.axis_index("c")
    s = lax.axis_index("s")
    wid = c * NS + s
    pltpu.sync_copy(dst_hbm.at[wid], dst_v)
    pltpu.sync_copy(w_hbm.at[wid], w_v)
    pltpu.sync_copy(z_hbm, acc1.at[pl.ds(s * STRIPE, STRIPE)])
    pltpu.sync_copy(z_hbm, acc2.at[pl.ds(s * STRIPE, STRIPE)])

    @pl.loop(0, BLK)
    def _(b):
        ones_v[b, :] = jnp.full((16,), 1.0, _f32)

    plsc.subcore_barrier()

    @pl.loop(0, NBLK)
    def _(j):
        @pl.loop(0, BLK, step=16)
        def _(q):
            wvec = w_v[j, pl.ds(q, 16)]
            for r in range(16):
                wb_v[q + r, :] = jnp.full((16,), 1.0, _f32) * wvec[r]

        pltpu.sync_copy(wb_v, acc1.at[dst_v.at[j]], add=True)
        pltpu.sync_copy(ones_v, acc2.at[dst_v.at[j]], add=True)

    plsc.subcore_barrier()
    sl = pl.ds(s * STRIPE, STRIPE)
    pltpu.sync_copy(acc1.at[sl], deg1_hbm.at[c, sl])
    pltpu.sync_copy(acc2.at[sl], deg2_hbm.at[c, sl])


CB = 8
NCH = NBLK // CB

_UNPACK_POS = np.empty((D1,), np.int32)
for _col in range(D1):
    _t, _j = _col // 32, _col % 32
    _UNPACK_POS[_col] = 32 * _t + (_j // 2 if _j % 2 == 0 else 16 + _j // 2)


@functools.partial(
    pl.kernel,
    mesh=_mesh,
    compiler_params=_sc_params_nl,
    out_type=jax.ShapeDtypeStruct((NC, ACC_ROWS, D1), _f32),
    scratch_types=[
        pltpu.VMEM((2 * CB, BLK), jnp.int32),
        pltpu.VMEM((2 * CB, BLK), jnp.int32),
        pltpu.VMEM((2 * CB, BLK), _f32),
        pltpu.VMEM((BLK, D1 // 2), jnp.int32),
        pltpu.VMEM((BLK, D1 // 2), jnp.int32),
        pltpu.VMEM((BLK, D1 // 2), jnp.int32),
        pltpu.VMEM((BLK, D1), _f32),
        pltpu.VMEM_SHARED((ACC_ROWS, D1), _f32),
        pltpu.SemaphoreType.DMA,
        pltpu.SemaphoreType.DMA,
        pltpu.SemaphoreType.DMA,
        pltpu.SemaphoreType.DMA,
        pltpu.SemaphoreType.DMA,
    ],
)
def _agg1_kernel(g_hbm, src_hbm, dst_hbm, w_hbm, z_hbm, out_hbm,
                 srcC, dstC, wC, g0, g1b, g2b, obuf,
                 acc, gs0, gs1, gs2, ssem, csem):
    c = lax.axis_index("c")
    s = lax.axis_index("s")
    wid = c * NS + s
    pltpu.sync_copy(z_hbm, acc.at[pl.ds(s * STRIPE, STRIPE)])

    gins = [g0, g1b, g2b]
    gsems = [gs0, gs1, gs2]

    def crow(j):
        return ((j // CB) % 2) * CB + (j % CB)

    def stage_start(ch):
        sl = pl.ds(ch * CB, CB)
        half = pl.ds((ch % 2) * CB, CB)
        pltpu.async_copy(src_hbm.at[wid, sl], srcC.at[half], csem)
        pltpu.async_copy(dst_hbm.at[wid, sl], dstC.at[half], csem)
        pltpu.async_copy(w_hbm.at[wid, sl], wC.at[half], csem)

    def stage_wait(ch):
        sl = pl.ds(ch * CB, CB)
        half = pl.ds((ch % 2) * CB, CB)
        pltpu.make_async_copy(src_hbm.at[wid, sl], srcC.at[half], csem).wait()
        pltpu.make_async_copy(dst_hbm.at[wid, sl], dstC.at[half], csem).wait()
        pltpu.make_async_copy(w_hbm.at[wid, sl], wC.at[half], csem).wait()

    def gather_start(j, k):
        pltpu.async_copy(g_hbm.at[srcC.at[crow(j)]], gins[k], gsems[k])

    def gather_wait(j, k):
        pltpu.make_async_copy(g_hbm.at[srcC.at[crow(j)]], gins[k],
                              gsems[k]).wait()

    def scatter_start(j):
        pltpu.async_copy(obuf, acc.at[dstC.at[crow(j)]], ssem, add=True)

    def scatter_wait(j):
        pltpu.make_async_copy(obuf, acc.at[dstC.at[crow(j)]], ssem).wait()

    def convert_scale(j, k):
        rw = crow(j)

        @pl.loop(0, BLK, step=16)
        def _(q):
            wvec = wC[rw, pl.ds(q, 16)]
            for r in range(16):
                wb = wvec[r]
                row = q + r
                for t in range(4):
                    u = gins[k][row, pl.ds(16 * t, 16)]
                    ab = plsc.bitcast(u, jnp.bfloat16)
                    lo, hi = plsc.unpack(ab, format=plsc.PackFormat.INTERLEAVED)
                    obuf[row, pl.ds(32 * t, 16)] = lo * wb
                    obuf[row, pl.ds(32 * t + 16, 16)] = hi * wb

    def manage_chunks(j):
        ch = j // CB

        @pl.when(jnp.logical_and(j % CB == 1,
                                 jnp.logical_and(j >= CB + 1, ch + 1 < NCH)))
        def _():
            stage_start(ch + 1)

        @pl.when(jnp.logical_and(j % CB == CB - 4, ch + 1 < NCH))
        def _():
            stage_wait(ch + 1)

    stage_start(0)
    stage_start(1)
    stage_wait(0)
    plsc.subcore_barrier()

    gather_start(0, 0)
    gather_start(1, 1)
    gather_wait(0, 0)
    gather_start(2, 2)
    convert_scale(0, 0)
    scatter_start(0)

    @pl.loop(1, NBLK - 4, step=3)
    def _(base):
        for kk in range(3):
            j = base + kk
            k = (1 + kk) % 3
            gather_wait(j, k)
            gather_start(j + 2, (k + 2) % 3)
            manage_chunks(j)
            scatter_wait(j - 1)
            convert_scale(j, k)
            scatter_start(j)

    for j in (NBLK - 4, NBLK - 3, NBLK - 2, NBLK - 1):
        k = j % 3
        gather_wait(j, k)
        if j + 2 < NBLK:
            gather_start(j + 2, (k + 2) % 3)
        scatter_wait(j - 1)
        convert_scale(j, k)
        scatter_start(j)
    scatter_wait(NBLK - 1)

    plsc.subcore_barrier()
    sl = pl.ds(s * STRIPE, STRIPE)
    pltpu.sync_copy(acc.at[sl], out_hbm.at[c, sl])


@functools.partial(
    pl.kernel,
    mesh=_mesh,
    compiler_params=_sc_params,
    out_type=jax.ShapeDtypeStruct((NC, ACC_ROWS, D2P), _f32),
    scratch_types=[
        pltpu.VMEM((NBLK, BLK), jnp.int32),
        pltpu.VMEM((NBLK, BLK), jnp.int32),
        pltpu.VMEM((BLK, D2P), _f32),
        pltpu.VMEM((BLK, D2P), _f32),
        pltpu.VMEM((BLK, D2P), _f32),
        pltpu.VMEM((BLK, D2P), _f32),
        pltpu.VMEM_SHARED((ACC_ROWS, D2P), _f32),
        pltpu.SemaphoreType.DMA,
        pltpu.SemaphoreType.DMA,
        pltpu.SemaphoreType.DMA,
        pltpu.SemaphoreType.DMA,
        pltpu.SemaphoreType.DMA,
        pltpu.SemaphoreType.DMA,
        pltpu.SemaphoreType.DMA,
        pltpu.SemaphoreType.DMA,
    ],
)
def _agg2_kernel(g_hbm, src_hbm, dst_hbm, z_hbm, out_hbm,
                 src_v, dst_v, b0, b1, b2, b3,
                 acc, gs0, gs1, gs2, gs3, ss0, ss1, ss2, ss3):
    c = lax.axis_index("c")
    s = lax.axis_index("s")
    wid = c * NS + s
    pltpu.sync_copy(src_hbm.at[wid], src_v)
    pltpu.sync_copy(dst_hbm.at[wid], dst_v)
    pltpu.sync_copy(z_hbm, acc.at[pl.ds(s * STRIPE, STRIPE)])
    plsc.subcore_barrier()

    bufs = [b0, b1, b2, b3]
    gsems = [gs0, gs1, gs2, gs3]
    ssems = [ss0, ss1, ss2, ss3]

    def gather_start(j, k):
        pltpu.async_copy(g_hbm.at[src_v.at[j]], bufs[k], gsems[k])

    def gather_wait(j, k):
        pltpu.make_async_copy(g_hbm.at[src_v.at[j]], bufs[k], gsems[k]).wait()

    def scatter_start(j, k):
        pltpu.async_copy(bufs[k], acc.at[dst_v.at[j]], ssems[k], add=True)

    def scatter_wait(j, k):
        pltpu.make_async_copy(bufs[k], acc.at[dst_v.at[j]], ssems[k]).wait()

    gather_start(0, 0)
    gather_start(1, 1)
    for j in (0, 1):
        gather_wait(j, j)
        scatter_start(j, j)
        gather_start(j + 2, j + 2)

    @pl.loop(2, NBLK - 2, step=4)
    def _(base):
        for kk in range(4):
            j = base + kk
            k = (2 + kk) % 4
            k2 = (k + 2) % 4
            gather_wait(j, k)
            scatter_start(j, k)
            scatter_wait(j - 2, k2)
            gather_start(j + 2, k2)

    for j in (NBLK - 2, NBLK - 1):
        k = j % 4
        gather_wait(j, k)
        scatter_start(j, k)
        scatter_wait(j - 2, (k + 2) % 4)
    scatter_wait(NBLK - 2, (NBLK - 2) % 4)
    scatter_wait(NBLK - 1, (NBLK - 1) % 4)

    plsc.subcore_barrier()
    sl = pl.ds(s * STRIPE, STRIPE)
    pltpu.sync_copy(acc.at[sl], out_hbm.at[c, sl])


RB = 2000
_G = N // RB


def _row_spec(d):
    return pl.BlockSpec((RB, d), lambda i: (i, 0))


def _acc_spec(d):
    return pl.BlockSpec((NC, RB, d), lambda i: (0, i, 0))


def _full_spec(r, d):
    return pl.BlockSpec((r, d), lambda i: (0, 0))


def _dinv(deg_ref, extra):
    d = deg_ref[0, :, 0:1] + deg_ref[1, :, 0:1] + extra
    return jnp.where(d > 0, lax.rsqrt(d), 0.0)


def _b_body(x_ref, w1_ref, deg1_ref, o_ref):
    h = jnp.dot(x_ref[...], w1_ref[...], precision=lax.Precision.HIGHEST,
                preferred_element_type=_f32)
    o_ref[...] = h * _dinv(deg1_ref, 1.0)


def _d_body(acc1_ref, g1_ref, deg1_ref, deg2_ref, w2_ref, b1_ref, o_ref):
    ssum = acc1_ref[0] + acc1_ref[1] + g1_ref[...]
    out1 = jnp.maximum(ssum * _dinv(deg1_ref, 1.0) + b1_ref[...], 0.0)
    h2 = jnp.dot(out1, w2_ref[...], precision=lax.Precision.HIGHEST,
                 preferred_element_type=_f32)
    o_ref[...] = h2 * _dinv(deg2_ref, 1.0)


def _f_body(acc2_ref, g2_ref, deg2_ref, b2_ref, o_ref):
    z = (acc2_ref[0] + acc2_ref[1] + g2_ref[...]) \
        * _dinv(deg2_ref, 1.0) + b2_ref[...]
    col = lax.broadcasted_iota(jnp.int32, (RB, D2P), 1)
    valid = col < D2
    m = jnp.max(jnp.where(valid, z, -jnp.inf), axis=1, keepdims=True)
    ex = jnp.where(valid, jnp.exp(z - m), 0.0)
    lse = jnp.log(jnp.sum(ex, axis=1, keepdims=True))
    o_ref[...] = (z - m - lse)[:, :D2]


def _tc_call(body, in_specs, out_d, out_shape=None):
    return pl.pallas_call(
        body,
        grid=(_G,),
        in_specs=in_specs,
        out_specs=_row_spec(out_d),
        out_shape=jax.ShapeDtypeStruct(out_shape or (N, out_d), _f32),
    )


def kernel(x, adj_t, edge_weight, W1, b1, W2, b2):
    src = adj_t[0].astype(jnp.int32)
    dst = adj_t[1].astype(jnp.int32)
    w = edge_weight.astype(_f32)

    pad = EP - E
    src3 = jnp.concatenate([src, jnp.zeros((pad,), jnp.int32)]).reshape(NW, NBLK, BLK)
    dst3 = jnp.concatenate([dst, jnp.full((pad,), N, jnp.int32)]).reshape(NW, NBLK, BLK)
    w3 = jnp.concatenate([w, jnp.zeros((pad,), _f32)]).reshape(NW, NBLK, BLK)

    z16 = jnp.zeros((STRIPE, 16), _f32)
    z128 = jnp.zeros((STRIPE, D1), _f32)
    z48 = jnp.zeros((STRIPE, D2P), _f32)

    w2p = jnp.concatenate([W2, jnp.zeros((D1, D2P - D2), _f32)], axis=1)
    b1r = b1.reshape(1, D1)
    b2p = jnp.concatenate([b2, jnp.zeros((D2P - D2,), _f32)]).reshape(1, D2P)

    deg1p, deg2p = _deg_kernel(dst3, w3, z16)
    g1 = _tc_call(
        _b_body, [_row_spec(D1), _full_spec(D1, D1), _acc_spec(16)], D1
    )(x, W1, deg1p)
    t1 = jax.lax.bitcast_convert_type(
        g1.astype(jnp.bfloat16).reshape(N, D1 // 2, 2), jnp.int32)
    acc1 = _agg1_kernel(t1, src3, dst3, w3, z128)
    acc1 = acc1[:, :, _UNPACK_POS]
    g2 = _tc_call(
        _d_body,
        [_acc_spec(D1), _row_spec(D1), _acc_spec(16), _acc_spec(16),
         _full_spec(D1, D2P), _full_spec(1, D1)],
        D2P,
    )(acc1, g1, deg1p, deg2p, w2p, b1r)
    acc2 = _agg2_kernel(g2, src3, dst3, z48)
    return _tc_call(
        _f_body,
        [_acc_spec(D2P), _row_spec(D2P), _acc_spec(16), _full_spec(1, D2P)],
        D2,
    )(acc2, g2, deg2p, b2p)
